# baseline (speedup 1.0000x reference)
import jax
import jax.numpy as jnp
from jax import lax
from jax.experimental import pallas as pl
from jax.experimental.pallas import tpu as pltpu
from jax.experimental.pallas import tpu_sc as plsc

NC = 2
NS = 16
NW = NC * NS

BATCH = 4
SEQ = 2048
HIDDEN = 2048
N = BATCH * SEQ
B_PER_W = N // NW
CHUNK = 16
NCHUNK = B_PER_W // CHUNK


def _sc_lookup(indices3, table):
    mesh = plsc.VectorSubcoreMesh(core_axis_name="c", subcore_axis_name="s")

    def body(idx_hbm, table_hbm, out_hbm, idx_v, buf0, buf1,
             gsem0, gsem1, ssem0, ssem1):
        wid = lax.axis_index("s") * NC + lax.axis_index("c")
        base = wid * B_PER_W
        pltpu.sync_copy(idx_hbm.at[wid], idx_v)

        bufs = (buf0, buf1)
        gsems = (gsem0, gsem1)
        ssems = (ssem0, ssem1)

        gathers = [None] * NCHUNK
        scatters = [None] * NCHUNK
        gathers[0] = pltpu.async_copy(
            table_hbm.at[idx_v.at[0]], bufs[0], gsems[0])
        for j in range(NCHUNK):
            p = j % 2
            if j + 1 < NCHUNK:
                if j - 1 >= 0:
                    scatters[j - 1].wait()
                gathers[j + 1] = pltpu.async_copy(
                    table_hbm.at[idx_v.at[j + 1]],
                    bufs[(j + 1) % 2], gsems[(j + 1) % 2])
            gathers[j].wait()
            scatters[j] = pltpu.async_copy(
                bufs[p], out_hbm.at[pl.ds(base + j * CHUNK, CHUNK)], ssems[p])
        scatters[NCHUNK - 2].wait()
        scatters[NCHUNK - 1].wait()

    run = pl.kernel(
        body,
        out_type=jax.ShapeDtypeStruct((N, HIDDEN), jnp.float32),
        mesh=mesh,
        scratch_types=[
            pltpu.VMEM((NCHUNK, CHUNK), jnp.int32),
            pltpu.VMEM((CHUNK, HIDDEN), jnp.float32),
            pltpu.VMEM((CHUNK, HIDDEN), jnp.float32),
            pltpu.SemaphoreType.DMA,
            pltpu.SemaphoreType.DMA,
            pltpu.SemaphoreType.DMA,
            pltpu.SemaphoreType.DMA,
        ],
    )
    return run(indices3, table)


def kernel(indices, table):
    idx3 = indices.astype(jnp.int32).reshape(NW, NCHUNK, CHUNK)
    out = _sc_lookup(idx3, table)
    return out.reshape(BATCH, SEQ, HIDDEN)


# speedup vs baseline: 1.0121x; 1.0121x over previous
import jax
import jax.numpy as jnp
from jax import lax
from jax.experimental import pallas as pl
from jax.experimental.pallas import tpu as pltpu
from jax.experimental.pallas import tpu_sc as plsc

NC = 2
NS = 16
NW = NC * NS

BATCH = 4
SEQ = 2048
HIDDEN = 2048
N = BATCH * SEQ
B_PER_W = N // NW
CHUNK = 16
NCHUNK = B_PER_W // CHUNK


def _sc_lookup(indices3, table):
    mesh = plsc.VectorSubcoreMesh(core_axis_name="c", subcore_axis_name="s")

    NBUF = 3

    def body(idx_hbm, table_hbm, out_hbm, idx_v, buf0, buf1, buf2,
             gsem0, gsem1, gsem2, ssem0, ssem1, ssem2):
        wid = lax.axis_index("s") * NC + lax.axis_index("c")
        base = wid * B_PER_W
        pltpu.sync_copy(idx_hbm.at[wid], idx_v)

        bufs = (buf0, buf1, buf2)
        gsems = (gsem0, gsem1, gsem2)
        ssems = (ssem0, ssem1, ssem2)

        gathers = [None] * NCHUNK
        scatters = [None] * NCHUNK
        for j in range(NBUF - 1):
            gathers[j] = pltpu.async_copy(
                table_hbm.at[idx_v.at[j]], bufs[j], gsems[j])
        for j in range(NCHUNK):
            p = j % NBUF
            if j + NBUF - 1 < NCHUNK:
                q = (j + NBUF - 1) % NBUF
                if j - 1 >= 0:
                    scatters[j - 1].wait()
                gathers[j + NBUF - 1] = pltpu.async_copy(
                    table_hbm.at[idx_v.at[j + NBUF - 1]], bufs[q], gsems[q])
            gathers[j].wait()
            scatters[j] = pltpu.async_copy(
                bufs[p], out_hbm.at[pl.ds(base + j * CHUNK, CHUNK)], ssems[p])
        for j in range(NCHUNK - NBUF, NCHUNK):
            scatters[j].wait()

    run = pl.kernel(
        body,
        out_type=jax.ShapeDtypeStruct((N, HIDDEN), jnp.float32),
        mesh=mesh,
        scratch_types=[
            pltpu.VMEM((NCHUNK, CHUNK), jnp.int32),
            pltpu.VMEM((CHUNK, HIDDEN), jnp.float32),
            pltpu.VMEM((CHUNK, HIDDEN), jnp.float32),
            pltpu.VMEM((CHUNK, HIDDEN), jnp.float32),
            pltpu.SemaphoreType.DMA,
            pltpu.SemaphoreType.DMA,
            pltpu.SemaphoreType.DMA,
            pltpu.SemaphoreType.DMA,
            pltpu.SemaphoreType.DMA,
            pltpu.SemaphoreType.DMA,
        ],
    )
    return run(indices3, table)


def kernel(indices, table):
    idx3 = indices.astype(jnp.int32).reshape(NW, NCHUNK, CHUNK)
    out = _sc_lookup(idx3, table)
    return out.reshape(BATCH, SEQ, HIDDEN)


# speedup vs baseline: 3.2759x; 3.2366x over previous
import jax
import jax.numpy as jnp
from jax import lax
from jax.experimental import pallas as pl
from jax.experimental.pallas import tpu as pltpu
from jax.experimental.pallas import tpu_sc as plsc

NC = 2
NS = 16
NW = NC * NS

BATCH = 4
SEQ = 2048
HIDDEN = 2048
N = BATCH * SEQ
B_PER_W = N // NW
G = 16
NG = B_PER_W // G


def _sc_lookup(indices2, table):
    mesh = plsc.VectorSubcoreMesh(core_axis_name="c", subcore_axis_name="s")

    def body(idx_hbm, table_hbm, out_hbm, idx_v, table_v, tsem, ssem0, ssem1):
        wid = lax.axis_index("s") * NC + lax.axis_index("c")
        base = wid * B_PER_W
        pltpu.sync_copy(idx_hbm.at[wid], idx_v)
        pltpu.async_copy(table_hbm, table_v, tsem).wait()

        ssems = (ssem0, ssem1)

        def fire(g, sem):
            ivec = idx_v[0, pl.ds(g * G, G)]
            for k in range(G):
                pltpu.async_copy(
                    table_v.at[pl.ds(ivec[k], 1)],
                    out_hbm.at[pl.ds(base + g * G + k, 1)],
                    sem)

        def drain(g, sem):
            blk = out_hbm.at[pl.ds(base + g * G, G)]
            pltpu.make_async_copy(blk, blk, sem).wait()

        fire(0, ssems[0])
        fire(1, ssems[1])

        def loop_body(gg, carry):
            g0 = 2 * gg
            drain(g0 - 2, ssems[0])
            fire(g0, ssems[0])
            drain(g0 - 1, ssems[1])
            fire(g0 + 1, ssems[1])
            return carry

        lax.fori_loop(1, NG // 2, loop_body, 0)
        drain(NG - 2, ssems[0])
        drain(NG - 1, ssems[1])

    run = pl.kernel(
        body,
        out_type=jax.ShapeDtypeStruct((N, HIDDEN), jnp.float32),
        mesh=mesh,
        scratch_types=[
            pltpu.VMEM((1, B_PER_W), jnp.int32),
            pltpu.VMEM((10, HIDDEN), jnp.float32),
            pltpu.SemaphoreType.DMA,
            pltpu.SemaphoreType.DMA,
            pltpu.SemaphoreType.DMA,
        ],
    )
    return run(indices2, table)


def kernel(indices, table):
    idx2 = indices.astype(jnp.int32).reshape(NW, 1, B_PER_W)
    out = _sc_lookup(idx2, table)
    return out.reshape(BATCH, SEQ, HIDDEN)
